# baseline (speedup 1.0000x reference)
import jax
import jax.numpy as jnp
from jax import lax
from jax.experimental import pallas as pl
from jax.experimental.pallas import tpu as pltpu
from jax.experimental.pallas import tpu_sc as plsc

N = 50000
NTOT = 2 * N
NS = 102400
PAD = NS - NTOT
C = 128
NGRAPH = 16
NXY = 32
PADKEY = NGRAPH * NXY * NXY
KHIST = 32768
RT = 128
NT = NS // RT
NW = 32
PERW = NS // NW
KROW = PERW // 128
SLW = KHIST // NW
NDUMP = 1024


def _cluster_body(xs_ref, ys_ref, b_ref, sub_ref):
    xs = xs_ref[...]
    ys = ys_ref[...]
    b = b_ref[...]
    big = jnp.float32(3.0e38)
    lox = jnp.zeros_like(xs)
    loy = jnp.zeros_like(ys)
    hix = jnp.zeros_like(xs)
    hiy = jnp.zeros_like(ys)
    for g in range(2 * NGRAPH):
        m = b == g
        minx = jnp.min(jnp.where(m, xs, big))
        maxx = jnp.max(jnp.where(m, xs, -big))
        miny = jnp.min(jnp.where(m, ys, big))
        maxy = jnp.max(jnp.where(m, ys, -big))
        lox = jnp.where(m, minx, lox)
        hix = jnp.where(m, maxx, hix)
        loy = jnp.where(m, miny, loy)
        hiy = jnp.where(m, maxy, hiy)
    nx = (xs - lox) / jnp.maximum(hix - lox, 1e-12)
    ny = (ys - loy) / jnp.maximum(hiy - loy, 1e-12)
    cx = jnp.clip(jnp.floor(nx * NXY).astype(jnp.int32), 0, NXY - 1)
    cy = jnp.clip(jnp.floor(ny * NXY).astype(jnp.int32), 0, NXY - 1)
    sub = (b % NGRAPH) * (NXY * NXY) + cy * NXY + cx
    sub_ref[...] = jnp.where(b < 2 * NGRAPH, sub, PADKEY)


def _enc_body(nv_ref, pos_ref, w1_ref, w2_ref, be_ref, h_ref):
    nv = nv_ref[...]
    px = pos_ref[:, 0:1]
    py = pos_ref[:, 1:2]
    h = jnp.dot(nv, w1_ref[...], preferred_element_type=jnp.float32)
    h = h + px * w2_ref[0:1, :] + py * w2_ref[1:2, :] + be_ref[...]
    h_ref[...] = h


def _band_body(hp_ref, hc_ref, hn_ref, axc_ref, tp_ref, tc_ref, tn_ref,
               wrel_ref, wroot_ref, brel_ref, out_ref):
    r = pl.program_id(0)
    sub_r = axc_ref[:, 2:3]
    px_r = axc_ref[:, 0:1]
    py_r = axc_ref[:, 1:2]
    rpos = r * RT + lax.broadcasted_iota(jnp.int32, (RT, 1), 0)
    acc = jnp.zeros((RT, C), jnp.float32)
    cnt = jnp.zeros((RT, 1), jnp.float32)
    blocks = (
        (hp_ref, tp_ref, jnp.maximum(r - 1, 0), r > 0),
        (hc_ref, tc_ref, r, True),
        (hn_ref, tn_ref, jnp.minimum(r + 1, NT - 1), r < NT - 1),
    )
    for h_ref, t_ref, cb, valid in blocks:
        t = t_ref[...]
        cpos = cb * RT + lax.broadcasted_iota(jnp.int32, (1, RT), 1)
        m = (sub_r == t[2:3, :]) & (rpos != cpos) & valid
        dx = px_r - t[0:1, :]
        dy = py_r - t[1:2, :]
        w = jnp.sqrt(dx * dx + dy * dy + 1e-12)
        wm = jnp.where(m, w, 0.0)
        acc = acc + jnp.dot(wm, h_ref[...], preferred_element_type=jnp.float32)
        cnt = cnt + jnp.sum(m.astype(jnp.float32), axis=1, keepdims=True)
    aggr = acc / jnp.maximum(cnt, 1.0)
    out = jnp.dot(aggr, wrel_ref[...], preferred_element_type=jnp.float32)
    out = out + jnp.dot(hc_ref[...], wroot_ref[...],
                        preferred_element_type=jnp.float32) + brel_ref[...]
    out_ref[...] = out


def _final_body(tnv_ref, buf_ref, wskip_ref, out_ref):
    out_ref[...] = buf_ref[...] + jnp.dot(
        tnv_ref[...], wskip_ref[...], preferred_element_type=jnp.float32)


def _wid():
    return lax.axis_index("s") * 2 + lax.axis_index("c")


def _hist_body(sub_hbm, hall_hbm, rank_hbm, keys_v, hist_v, rank_v):
    wid = _wid()
    pltpu.sync_copy(sub_hbm.at[pl.ds(wid * PERW, PERW)],
                    keys_v.at[pl.ds(16, PERW)])
    neg16 = jnp.full((16,), -1, jnp.int32)
    keys_v[pl.ds(0, 16)] = neg16
    keys_v[pl.ds(16 + PERW, 16)] = neg16
    z16 = jnp.zeros((16,), jnp.int32)

    def zero_step(i, c):
        hist_v[pl.ds(i * 16, 16)] = z16
        return c

    lax.fori_loop(0, KHIST // 16, zero_step, 0)

    lane = lax.iota(jnp.int32, 16)

    def step(i, c):
        o = 16 + i * 16
        k16 = keys_v[pl.ds(o, 16)]
        base16 = plsc.load_gather(hist_v, [k16])
        prior = jnp.zeros((16,), jnp.int32)
        later = jnp.zeros((16,), jnp.int32)
        for s in range(1, 16):
            shb = keys_v[pl.ds(o - s, 16)]
            prior = prior + jnp.where((lane >= s) & (k16 == shb), 1, 0)
            shf = keys_v[pl.ds(o + s, 16)]
            later = later + jnp.where((lane < 16 - s) & (k16 == shf), 1, 0)
        rank_v[pl.ds(i * 16, 16)] = base16 + prior
        plsc.store_scatter(hist_v, [k16], base16 + prior + 1,
                           mask=later == 0)
        return c

    lax.fori_loop(0, PERW // 16, step, 0)
    pltpu.sync_copy(hist_v, hall_hbm.at[wid])
    pltpu.sync_copy(rank_v, rank_hbm.at[pl.ds(wid * PERW, PERW)])


def _off_body(hall_hbm, opart_hbm, tvec_hbm, col_v, obuf_v, tvb_v):
    wid = _wid()
    pltpu.sync_copy(hall_hbm.at[:, pl.ds(wid * SLW, SLW)], col_v)

    def chunk(i, carry):
        tot = jnp.zeros((16,), jnp.int32)
        for w in range(NW):
            tot = tot + col_v[w, pl.ds(i * 16, 16)]
        ex16 = plsc.cumsum(tot) - tot + carry
        run = ex16
        for w in range(NW):
            obuf_v[w, pl.ds(i * 16, 16)] = run
            run = run + col_v[w, pl.ds(i * 16, 16)]
        return carry + jnp.sum(tot)

    total = lax.fori_loop(0, SLW // 16, chunk, jnp.int32(0))
    pltpu.sync_copy(obuf_v, opart_hbm.at[:, pl.ds(wid * SLW, SLW)])
    tvb_v[...] = jnp.zeros((16,), jnp.int32) + total
    pltpu.sync_copy(tvb_v, tvec_hbm.at[wid])


def _move_body(h_hbm, px_hbm, py_hbm, sub_hbm, rank_hbm, opart_hbm, tvec_hbm,
               hs_hbm, pxs_hbm, pys_hbm, subi_hbm, sidx_hbm,
               keys_v, rank_v, off_v, t_v, spref_v, pos_v, sval_v,
               px_v, py_v, hrow2_v, seml, sems, seme):
    wid = _wid()
    pltpu.sync_copy(sub_hbm.at[pl.ds(wid * PERW, PERW)], keys_v)
    pltpu.sync_copy(rank_hbm.at[pl.ds(wid * PERW, PERW)], rank_v)
    pltpu.sync_copy(px_hbm.at[pl.ds(wid * PERW, PERW)], px_v)
    pltpu.sync_copy(py_hbm.at[pl.ds(wid * PERW, PERW)], py_v)
    pltpu.sync_copy(opart_hbm.at[wid], off_v)
    pltpu.sync_copy(tvec_hbm, t_v)

    lane = lax.iota(jnp.int32, 16)
    carry = jnp.zeros((16,), jnp.int32)
    v0 = jnp.zeros((16,), jnp.int32)
    v1 = jnp.zeros((16,), jnp.int32)
    for w in range(NW):
        tw = t_v[w, :]
        if w < 16:
            v0 = jnp.where(lane == w, carry, v0)
        else:
            v1 = jnp.where(lane == (w - 16), carry, v1)
        carry = carry + tw
    spref_v[pl.ds(0, 16)] = v0
    spref_v[pl.ds(16, 16)] = v1

    for j in range(KROW):
        for i in range(8):
            o = j * 128 + i * 16
            k16 = keys_v[pl.ds(o, 16)]
            o16 = plsc.load_gather(off_v, [k16])
            s16 = plsc.load_gather(spref_v, [k16 >> 10])
            pos16 = o16 + s16 + rank_v[pl.ds(o, 16)]
            pos_v[j, pl.ds(i * 16, 16)] = pos16
            n16 = wid * PERW + o + lane
            is_tgt = (n16 >= N) & (n16 < NTOT)
            sval_v[j, pl.ds(i * 16, 16)] = jnp.where(
                is_tgt, n16 - N, N + (n16 & (NDUMP - 1)))

    pltpu.sync_copy(h_hbm.at[pl.ds(wid * PERW, 128)], hrow2_v.at[0])
    hdesc = [None, None]
    edescs = []
    for j in range(KROW):
        cur = j & 1
        nxt = cur ^ 1
        ld = None
        if j + 1 < KROW:
            if hdesc[nxt] is not None:
                hdesc[nxt].wait()
                hdesc[nxt] = None
            ld = pltpu.async_copy(
                h_hbm.at[pl.ds(wid * PERW + (j + 1) * 128, 128)],
                hrow2_v.at[nxt], seml)
        idx = pos_v.at[j]
        hdesc[cur] = pltpu.async_copy(hrow2_v.at[cur], hs_hbm.at[idx], sems)
        sl = pl.ds(j * 128, 128)
        edescs.append(pltpu.async_copy(px_v.at[sl], pxs_hbm.at[idx], seme))
        edescs.append(pltpu.async_copy(py_v.at[sl], pys_hbm.at[idx], seme))
        edescs.append(pltpu.async_copy(keys_v.at[sl], subi_hbm.at[idx], seme))
        edescs.append(pltpu.async_copy(sval_v.at[j], sidx_hbm.at[idx], seme))
        if ld is not None:
            ld.wait()
        if len(edescs) >= 8:
            for d in edescs:
                d.wait()
            edescs = []
    for d in hdesc:
        if d is not None:
            d.wait()
    for d in edescs:
        d.wait()


def _scatter_body(outs_hbm, sidx_hbm, buf_hbm, idx_v, row2_v, seml, sems):
    wid = _wid()
    pltpu.sync_copy(sidx_hbm.at[wid], idx_v)
    pltpu.sync_copy(outs_hbm.at[pl.ds(wid * PERW, 128)], row2_v.at[0])
    sdesc = [None, None]
    for j in range(KROW):
        cur = j & 1
        nxt = cur ^ 1
        ld = None
        if j + 1 < KROW:
            if sdesc[nxt] is not None:
                sdesc[nxt].wait()
                sdesc[nxt] = None
            ld = pltpu.async_copy(
                outs_hbm.at[pl.ds(wid * PERW + (j + 1) * 128, 128)],
                row2_v.at[nxt], seml)
        sdesc[cur] = pltpu.async_copy(row2_v.at[cur], buf_hbm.at[idx_v.at[j]],
                                      sems)
        if ld is not None:
            ld.wait()
    for d in sdesc:
        if d is not None:
            d.wait()


def kernel(src_node_values, src_coords, src_batch, tgt_node_values, tgt_coords,
           tgt_batch, W_enc, b_enc, W_rel, b_rel, W_root, W_skip):
    f32 = jnp.float32
    i32 = jnp.int32

    coords = jnp.concatenate(
        [src_coords, tgt_coords, jnp.zeros((PAD, 2), f32)], axis=0)
    batch32 = jnp.concatenate(
        [src_batch, tgt_batch + NGRAPH, jnp.full((PAD,), 2 * NGRAPH, i32)])
    nv = jnp.concatenate(
        [src_node_values, tgt_node_values, jnp.zeros((PAD, C), f32)], axis=0)

    sub2 = pl.pallas_call(
        _cluster_body,
        out_shape=jax.ShapeDtypeStruct((128, NS // 128), i32),
    )(coords[:, 0].reshape(128, NS // 128),
      coords[:, 1].reshape(128, NS // 128),
      batch32.reshape(128, NS // 128))
    subp = sub2.reshape(NS)

    h = pl.pallas_call(
        _enc_body,
        grid=(NS // 512,),
        in_specs=[
            pl.BlockSpec((512, C), lambda r: (r, 0)),
            pl.BlockSpec((512, 2), lambda r: (r, 0)),
            pl.BlockSpec((C, C), lambda r: (0, 0)),
            pl.BlockSpec((2, C), lambda r: (0, 0)),
            pl.BlockSpec((1, C), lambda r: (0, 0)),
        ],
        out_specs=pl.BlockSpec((512, C), lambda r: (r, 0)),
        out_shape=jax.ShapeDtypeStruct((NS, C), f32),
    )(nv, coords, W_enc[:C], W_enc[C:C + 2], b_enc.reshape(1, C))

    mesh = plsc.VectorSubcoreMesh(core_axis_name="c", subcore_axis_name="s",
                                  num_cores=2, num_subcores=16)
    scparams = pltpu.CompilerParams(needs_layout_passes=False)
    hall, rank = pl.kernel(
        _hist_body,
        compiler_params=scparams,
        out_type=[jax.ShapeDtypeStruct((NW, KHIST), i32),
                  jax.ShapeDtypeStruct((NS,), i32)],
        mesh=mesh,
        scratch_types=[pltpu.VMEM((PERW + 32,), i32),
                       pltpu.VMEM((KHIST,), i32),
                       pltpu.VMEM((PERW,), i32)],
    )(subp)

    opart, tvec = pl.kernel(
        _off_body,
        compiler_params=scparams,
        out_type=[jax.ShapeDtypeStruct((NW, KHIST), i32),
                  jax.ShapeDtypeStruct((NW, 16), i32)],
        mesh=mesh,
        scratch_types=[pltpu.VMEM((NW, SLW), i32),
                       pltpu.VMEM((NW, SLW), i32),
                       pltpu.VMEM((16,), i32)],
    )(hall)

    hs, pxs, pys, subi, sidx = pl.kernel(
        _move_body,
        compiler_params=scparams,
        out_type=[jax.ShapeDtypeStruct((NS, C), f32),
                  jax.ShapeDtypeStruct((NS,), f32),
                  jax.ShapeDtypeStruct((NS,), f32),
                  jax.ShapeDtypeStruct((NS,), i32),
                  jax.ShapeDtypeStruct((NS,), i32)],
        mesh=mesh,
        scratch_types=[pltpu.VMEM((PERW,), i32),
                       pltpu.VMEM((PERW,), i32),
                       pltpu.VMEM((KHIST,), i32),
                       pltpu.VMEM((NW, 16), i32),
                       pltpu.VMEM((NW,), i32),
                       pltpu.VMEM((KROW, 128), i32),
                       pltpu.VMEM((KROW, 128), i32),
                       pltpu.VMEM((PERW,), f32),
                       pltpu.VMEM((PERW,), f32),
                       pltpu.VMEM((2, 128, C), f32),
                       pltpu.SemaphoreType.DMA,
                       pltpu.SemaphoreType.DMA,
                       pltpu.SemaphoreType.DMA],
    )(h, coords[:, 0], coords[:, 1], subp, rank, opart, tvec)

    subf = subi.astype(f32)
    axc = jnp.concatenate([pxs[:, None], pys[:, None], subf[:, None]], axis=1)
    auxT = jnp.concatenate([pxs[None], pys[None], subf[None]], axis=0)
    outs = pl.pallas_call(
        _band_body,
        grid=(NT,),
        in_specs=[
            pl.BlockSpec((RT, C), lambda r: (jnp.maximum(r - 1, 0), 0)),
            pl.BlockSpec((RT, C), lambda r: (r, 0)),
            pl.BlockSpec((RT, C), lambda r: (jnp.minimum(r + 1, NT - 1), 0)),
            pl.BlockSpec((RT, 3), lambda r: (r, 0)),
            pl.BlockSpec((3, RT), lambda r: (0, jnp.maximum(r - 1, 0))),
            pl.BlockSpec((3, RT), lambda r: (0, r)),
            pl.BlockSpec((3, RT), lambda r: (0, jnp.minimum(r + 1, NT - 1))),
            pl.BlockSpec((C, C), lambda r: (0, 0)),
            pl.BlockSpec((C, C), lambda r: (0, 0)),
            pl.BlockSpec((1, C), lambda r: (0, 0)),
        ],
        out_specs=pl.BlockSpec((RT, C), lambda r: (r, 0)),
        out_shape=jax.ShapeDtypeStruct((NS, C), f32),
    )(hs, hs, hs, axc, auxT, auxT, auxT, W_rel, W_root, b_rel.reshape(1, C))

    buf = pl.kernel(
        _scatter_body,
        out_type=jax.ShapeDtypeStruct((N + NDUMP, C), f32),
        mesh=mesh,
        scratch_types=[pltpu.VMEM((KROW, 128), i32),
                       pltpu.VMEM((2, 128, C), f32),
                       pltpu.SemaphoreType.DMA,
                       pltpu.SemaphoreType.DMA],
    )(outs, sidx.reshape(NW, KROW, 128))

    tgt_values = pl.pallas_call(
        _final_body,
        grid=(125,),
        in_specs=[
            pl.BlockSpec((400, C), lambda r: (r, 0)),
            pl.BlockSpec((400, C), lambda r: (r, 0)),
            pl.BlockSpec((C, C), lambda r: (0, 0)),
        ],
        out_specs=pl.BlockSpec((400, C), lambda r: (r, 0)),
        out_shape=jax.ShapeDtypeStruct((N, C), f32),
    )(tgt_node_values, buf, W_skip)
    return tgt_values


# speedup vs baseline: 1.2575x; 1.2575x over previous
import jax
import jax.numpy as jnp
from jax import lax
from jax.experimental import pallas as pl
from jax.experimental.pallas import tpu as pltpu
from jax.experimental.pallas import tpu_sc as plsc

N = 50000
NTOT = 2 * N
NS = 102400
PAD = NS - NTOT
C = 128
NGRAPH = 16
NXY = 32
BIG = 1 << 30
RT = 128
NT = NS // RT
NW = 32
PERW = NS // NW
KROW = PERW // 128
NDUMP = 1024


def _cluster_body(xs_ref, ys_ref, b_ref, sub_ref):
    xs = xs_ref[...]
    ys = ys_ref[...]
    b = b_ref[...]
    big = jnp.float32(3.0e38)
    lox = jnp.zeros_like(xs)
    loy = jnp.zeros_like(ys)
    hix = jnp.zeros_like(xs)
    hiy = jnp.zeros_like(ys)
    for g in range(2 * NGRAPH):
        m = b == g
        minx = jnp.min(jnp.where(m, xs, big))
        maxx = jnp.max(jnp.where(m, xs, -big))
        miny = jnp.min(jnp.where(m, ys, big))
        maxy = jnp.max(jnp.where(m, ys, -big))
        lox = jnp.where(m, minx, lox)
        hix = jnp.where(m, maxx, hix)
        loy = jnp.where(m, miny, loy)
        hiy = jnp.where(m, maxy, hiy)
    nx = (xs - lox) / jnp.maximum(hix - lox, 1e-12)
    ny = (ys - loy) / jnp.maximum(hiy - loy, 1e-12)
    cx = jnp.clip(jnp.floor(nx * NXY).astype(jnp.int32), 0, NXY - 1)
    cy = jnp.clip(jnp.floor(ny * NXY).astype(jnp.int32), 0, NXY - 1)
    sub = (b % NGRAPH) * (NXY * NXY) + cy * NXY + cx
    sub_ref[...] = jnp.where(b < 2 * NGRAPH, sub, BIG)


def _enc_body(nv_ref, pos_ref, w1_ref, w2_ref, be_ref, h_ref):
    nv = nv_ref[...]
    px = pos_ref[:, 0:1]
    py = pos_ref[:, 1:2]
    h = jnp.dot(nv, w1_ref[...], preferred_element_type=jnp.float32)
    h = h + px * w2_ref[0:1, :] + py * w2_ref[1:2, :] + be_ref[...]
    h_ref[...] = h


def _band_body(hp_ref, hc_ref, hn_ref, ap_ref, ac_ref, an_ref,
               wrel_ref, wroot_ref, brel_ref, perm_ref, out_ref, sidx_ref):
    r = pl.program_id(0)
    axc = ac_ref[...]
    sub_r = axc[:, 2:3]
    px_r = axc[:, 0:1]
    py_r = axc[:, 1:2]
    rpos = r * RT + lax.broadcasted_iota(jnp.int32, (RT, 1), 0)
    acc = jnp.zeros((RT, C), jnp.float32)
    cnt = jnp.zeros((RT, 1), jnp.float32)
    blocks = (
        (hp_ref, ap_ref, jnp.maximum(r - 1, 0), r > 0),
        (hc_ref, ac_ref, r, True),
        (hn_ref, an_ref, jnp.minimum(r + 1, NT - 1), r < NT - 1),
    )
    for h_ref, a_ref, cb, valid in blocks:
        t = jnp.transpose(a_ref[:, 0:3], (1, 0))
        cpos = cb * RT + lax.broadcasted_iota(jnp.int32, (1, RT), 1)
        m = (sub_r == t[2:3, :]) & (rpos != cpos) & valid
        dx = px_r - t[0:1, :]
        dy = py_r - t[1:2, :]
        w = jnp.sqrt(dx * dx + dy * dy + 1e-12)
        wm = jnp.where(m, w, 0.0)
        acc = acc + jnp.dot(wm, h_ref[...], preferred_element_type=jnp.float32)
        cnt = cnt + jnp.sum(m.astype(jnp.float32), axis=1, keepdims=True)
    aggr = acc / jnp.maximum(cnt, 1.0)
    out = jnp.dot(aggr, wrel_ref[...], preferred_element_type=jnp.float32)
    out = out + jnp.dot(hc_ref[...], wroot_ref[...],
                        preferred_element_type=jnp.float32) + brel_ref[...]
    out_ref[...] = out
    pp = perm_ref[...]
    ii = lax.broadcasted_iota(jnp.int32, (1, 1, RT), 2)
    dump = N + ((r * RT + ii) & (NDUMP - 1))
    is_tgt = (pp >= N) & (pp < NTOT)
    sidx_ref[...] = jnp.where(is_tgt, pp - N, dump)


def _final_body(tnv_ref, buf_ref, wskip_ref, out_ref):
    out_ref[...] = buf_ref[...] + jnp.dot(
        tnv_ref[...], wskip_ref[...], preferred_element_type=jnp.float32)


def _wid():
    return lax.axis_index("s") * 2 + lax.axis_index("c")


def _gather_body(h_hbm, aux_hbm, perm_hbm, hs_hbm, auxs_hbm,
                 idx_v, hrow2_v, arow2_v, semh, sema):
    wid = _wid()
    pltpu.sync_copy(perm_hbm.at[wid], idx_v)
    descs = [None, None]

    def fire(j, buf):
        dh = pltpu.async_copy(h_hbm.at[idx_v.at[j]], hrow2_v.at[buf], semh)
        da = pltpu.async_copy(aux_hbm.at[idx_v.at[j]], arow2_v.at[buf], sema)
        return (dh, da)

    descs[0] = fire(0, 0)
    for j in range(KROW):
        cur = j & 1
        nxt = cur ^ 1
        dh, da = descs[cur]
        dh.wait()
        da.wait()
        if j + 1 < KROW:
            descs[nxt] = fire(j + 1, nxt)
        base = wid * PERW + j * 128
        pltpu.sync_copy(hrow2_v.at[cur], hs_hbm.at[pl.ds(base, 128)])
        pltpu.sync_copy(arow2_v.at[cur], auxs_hbm.at[pl.ds(base, 128)])


def _scatter_body(outs_hbm, sidx_hbm, buf_hbm, idx_v, row2_v, seml, sems):
    wid = _wid()
    pltpu.sync_copy(sidx_hbm.at[wid], idx_v)
    pltpu.sync_copy(outs_hbm.at[pl.ds(wid * PERW, 128)], row2_v.at[0])
    sdesc = [None, None]
    for j in range(KROW):
        cur = j & 1
        nxt = cur ^ 1
        ld = None
        if j + 1 < KROW:
            if sdesc[nxt] is not None:
                sdesc[nxt].wait()
                sdesc[nxt] = None
            ld = pltpu.async_copy(
                outs_hbm.at[pl.ds(wid * PERW + (j + 1) * 128, 128)],
                row2_v.at[nxt], seml)
        sdesc[cur] = pltpu.async_copy(row2_v.at[cur], buf_hbm.at[idx_v.at[j]],
                                      sems)
        if ld is not None:
            ld.wait()
    for d in sdesc:
        if d is not None:
            d.wait()


def kernel(src_node_values, src_coords, src_batch, tgt_node_values, tgt_coords,
           tgt_batch, W_enc, b_enc, W_rel, b_rel, W_root, W_skip):
    f32 = jnp.float32
    i32 = jnp.int32

    coords = jnp.concatenate(
        [src_coords, tgt_coords, jnp.zeros((PAD, 2), f32)], axis=0)
    batch32 = jnp.concatenate(
        [src_batch, tgt_batch + NGRAPH, jnp.full((PAD,), 2 * NGRAPH, i32)])
    nv = jnp.concatenate(
        [src_node_values, tgt_node_values, jnp.zeros((PAD, C), f32)], axis=0)

    sub2 = pl.pallas_call(
        _cluster_body,
        out_shape=jax.ShapeDtypeStruct((128, NS // 128), i32),
    )(coords[:, 0].reshape(128, NS // 128),
      coords[:, 1].reshape(128, NS // 128),
      batch32.reshape(128, NS // 128))
    subp = sub2.reshape(NS)

    perm = jnp.argsort(subp).astype(i32)

    h = pl.pallas_call(
        _enc_body,
        grid=(NS // 512,),
        in_specs=[
            pl.BlockSpec((512, C), lambda r: (r, 0)),
            pl.BlockSpec((512, 2), lambda r: (r, 0)),
            pl.BlockSpec((C, C), lambda r: (0, 0)),
            pl.BlockSpec((2, C), lambda r: (0, 0)),
            pl.BlockSpec((1, C), lambda r: (0, 0)),
        ],
        out_specs=pl.BlockSpec((512, C), lambda r: (r, 0)),
        out_shape=jax.ShapeDtypeStruct((NS, C), f32),
    )(nv, coords, W_enc[:C], W_enc[C:C + 2], b_enc.reshape(1, C))

    aux = jnp.concatenate(
        [coords, subp.astype(f32)[:, None], jnp.zeros((NS, 125), f32)], axis=1)
    mesh = plsc.VectorSubcoreMesh(core_axis_name="c", subcore_axis_name="s",
                                  num_cores=2, num_subcores=16)
    hs, auxs = pl.kernel(
        _gather_body,
        out_type=[jax.ShapeDtypeStruct((NS, C), f32),
                  jax.ShapeDtypeStruct((NS, 128), f32)],
        mesh=mesh,
        scratch_types=[pltpu.VMEM((KROW, 128), i32),
                       pltpu.VMEM((2, 128, C), f32),
                       pltpu.VMEM((2, 128, 128), f32),
                       pltpu.SemaphoreType.DMA,
                       pltpu.SemaphoreType.DMA],
    )(h, aux, perm.reshape(NW, KROW, 128))

    outs, sidx3 = pl.pallas_call(
        _band_body,
        grid=(NT,),
        in_specs=[
            pl.BlockSpec((RT, C), lambda r: (jnp.maximum(r - 1, 0), 0)),
            pl.BlockSpec((RT, C), lambda r: (r, 0)),
            pl.BlockSpec((RT, C), lambda r: (jnp.minimum(r + 1, NT - 1), 0)),
            pl.BlockSpec((RT, 128), lambda r: (jnp.maximum(r - 1, 0), 0)),
            pl.BlockSpec((RT, 128), lambda r: (r, 0)),
            pl.BlockSpec((RT, 128), lambda r: (jnp.minimum(r + 1, NT - 1), 0)),
            pl.BlockSpec((C, C), lambda r: (0, 0)),
            pl.BlockSpec((C, C), lambda r: (0, 0)),
            pl.BlockSpec((1, C), lambda r: (0, 0)),
            pl.BlockSpec((1, 1, RT), lambda r: (r, 0, 0)),
        ],
        out_specs=[
            pl.BlockSpec((RT, C), lambda r: (r, 0)),
            pl.BlockSpec((1, 1, RT), lambda r: (r, 0, 0)),
        ],
        out_shape=[jax.ShapeDtypeStruct((NS, C), f32),
                   jax.ShapeDtypeStruct((NT, 1, RT), i32)],
    )(hs, hs, hs, auxs, auxs, auxs, W_rel, W_root, b_rel.reshape(1, C),
      perm.reshape(NT, 1, RT))

    buf = pl.kernel(
        _scatter_body,
        out_type=jax.ShapeDtypeStruct((N + NDUMP, C), f32),
        mesh=mesh,
        scratch_types=[pltpu.VMEM((KROW, 128), i32),
                       pltpu.VMEM((2, 128, C), f32),
                       pltpu.SemaphoreType.DMA,
                       pltpu.SemaphoreType.DMA],
    )(outs, sidx3.reshape(NW, KROW, 128))

    tgt_values = pl.pallas_call(
        _final_body,
        grid=(125,),
        in_specs=[
            pl.BlockSpec((400, C), lambda r: (r, 0)),
            pl.BlockSpec((400, C), lambda r: (r, 0)),
            pl.BlockSpec((C, C), lambda r: (0, 0)),
        ],
        out_specs=pl.BlockSpec((400, C), lambda r: (r, 0)),
        out_shape=jax.ShapeDtypeStruct((N, C), f32),
    )(tgt_node_values, buf, W_skip)
    return tgt_values
